# baseline (speedup 1.0000x reference)
import functools

import jax
import jax.numpy as jnp
from jax.experimental import pallas as pl
from jax.experimental.pallas import tpu as pltpu

D_MODEL = 128
BLK = 8192


def _tc_gather(idx2d, table_pad, n_total):
    grid = (n_total // BLK,)

    def body(idx_ref, table_ref, out_ref):
        idx = idx_ref[...]
        cols = jax.lax.broadcasted_iota(jnp.int32, (BLK, 128), 1)
        oh = (idx == cols).astype(jnp.float32)
        out_ref[...] = jnp.dot(
            oh, table_ref[...], preferred_element_type=jnp.float32
        )

    return pl.pallas_call(
        body,
        grid=grid,
        in_specs=[
            pl.BlockSpec((BLK, 1), lambda i: (i, 0)),
            pl.BlockSpec((128, D_MODEL), lambda i: (0, 0)),
        ],
        out_specs=pl.BlockSpec((BLK, D_MODEL), lambda i: (i, 0)),
        out_shape=jax.ShapeDtypeStruct((n_total, D_MODEL), jnp.float32),
        compiler_params=pltpu.CompilerParams(
            dimension_semantics=("arbitrary",),
        ),
    )(idx2d, table_pad)


def kernel(cumulative_positions, position_embeddings):
    b, s = cumulative_positions.shape
    n_total = b * s
    idx2d = cumulative_positions.reshape(n_total, 1).astype(jnp.int32)
    table_pad = jnp.zeros((128, D_MODEL), jnp.float32).at[:51].set(position_embeddings)
    out = _tc_gather(idx2d, table_pad, n_total)
    return out.reshape(b, s, D_MODEL)
